# Initial kernel scaffold; baseline (speedup 1.0000x reference)
import jax
import jax.numpy as jnp
from jax.experimental import pallas as pl


def kernel(x, nbr_0, nbr_1, trans, W_ih_0, W_hh_0, b_ih_0, b_hh_0, W_self_0, b_self_0, W_neigh_0, W_ih_1, W_hh_1, b_ih_1, b_hh_1, W_self_1, b_self_1, W_neigh_1):
    raise NotImplementedError("write your pallas kernel here")


# speedup vs baseline: 3.1271x; 3.1271x over previous
import functools

import jax
import jax.numpy as jnp
from jax import lax
from jax.experimental import pallas as pl
from jax.experimental.pallas import tpu as pltpu
from jax.experimental.pallas import tpu_sc as plsc

N = 10000
DEG = 32
D = 128


_CH = 128
_ROWS = N * DEG
_NCHUNK = _ROWS // _CH


def _sc_gather(x, idx0, idx1):
    info = plsc.get_sparse_core_info()
    nw = info.num_cores * info.num_subcores
    nfull = _NCHUNK // nw
    extra = _NCHUNK % nw

    mesh = plsc.VectorSubcoreMesh(core_axis_name="c", subcore_axis_name="s")

    @functools.partial(
        pl.kernel,
        mesh=mesh,
        out_type=[jax.ShapeDtypeStruct((_ROWS, D), jnp.float32)] * 2,
        scratch_types=[
            pltpu.VMEM((_CH,), jnp.int32),
            pltpu.VMEM((_CH, D), jnp.float32),
            pltpu.SemaphoreType.DMA,
        ],
    )
    def gather(x_hbm, i0_hbm, i1_hbm, o0_hbm, o1_hbm, idx_v, rows_v, sem):
        wid = lax.axis_index("s") * info.num_cores + lax.axis_index("c")
        n_w = nfull + jnp.where(wid < extra, 1, 0)

        def one_rel(idx_hbm, out_hbm):
            def body(i, carry):
                base = (wid + i * nw) * _CH
                pltpu.sync_copy(idx_hbm.at[pl.ds(base, _CH)], idx_v)
                pltpu.async_copy(x_hbm.at[idx_v], rows_v, sem).wait()
                pltpu.sync_copy(rows_v, out_hbm.at[pl.ds(base, _CH)])
                return carry

            lax.fori_loop(0, n_w, body, 0)

        one_rel(i0_hbm, o0_hbm)
        one_rel(i1_hbm, o1_hbm)

    return gather(x, idx0, idx1)


_BLK = 400
_GRID = N // _BLK


def _tc_body(x_ref, f0_ref, f1_ref,
             wcat0_ref, bc0_ref, ws0_ref, bs0_ref, wn0_ref,
             wcat1_ref, bc1_ref, ws1_ref, bs1_ref, wn1_ref,
             trans_ref, out_ref):
    xb = x_ref[...]
    hs = []
    for f_ref, wcat_ref, bc_ref, ws_ref, bs_ref, wn_ref in (
        (f0_ref, wcat0_ref, bc0_ref, ws0_ref, bs0_ref, wn0_ref),
        (f1_ref, wcat1_ref, bc1_ref, ws1_ref, bs1_ref, wn1_ref),
    ):
        wcat = wcat_ref[...]
        bc = bc_ref[...]

        def step(t, hc, f_ref=f_ref, wcat=wcat, bc=bc):
            h, c = hc
            xt = f_ref[t]
            xh = jnp.concatenate([xt, h], axis=1)
            g = jnp.dot(xh, wcat, preferred_element_type=jnp.float32) + bc
            i_ = jax.nn.sigmoid(g[:, 0:D])
            fg = jax.nn.sigmoid(g[:, D:2 * D])
            gg = jnp.tanh(g[:, 2 * D:3 * D])
            og = jax.nn.sigmoid(g[:, 3 * D:4 * D])
            c2 = fg * c + i_ * gg
            h2 = og * jnp.tanh(c2)
            return (h2, c2)

        z = jnp.zeros((_BLK, D), jnp.float32)
        h_fin, _ = lax.fori_loop(0, DEG, step, (z, z))
        rst = (jnp.dot(xb, ws_ref[...], preferred_element_type=jnp.float32)
               + bs_ref[...]
               + jnp.dot(h_fin, wn_ref[...], preferred_element_type=jnp.float32))
        hs.append(jnp.where(rst > 0, rst, 0.01 * rst))

    h0, h1 = hs
    tr = trans_ref[...]
    a0 = jnp.sum(jnp.dot(h0, tr, preferred_element_type=jnp.float32) * xb,
                 axis=1, keepdims=True)
    a1 = jnp.sum(jnp.dot(h1, tr, preferred_element_type=jnp.float32) * xb,
                 axis=1, keepdims=True)
    m = jnp.maximum(a0, a1)
    e0 = jnp.exp(a0 - m)
    e1 = jnp.exp(a1 - m)
    out_ref[...] = (e0 * h0 + e1 * h1) / (e0 + e1)


def _full(shape):
    return pl.BlockSpec(shape, lambda i: tuple(0 for _ in shape))


def kernel(x, nbr_0, nbr_1, trans,
           W_ih_0, W_hh_0, b_ih_0, b_hh_0, W_self_0, b_self_0, W_neigh_0,
           W_ih_1, W_hh_1, b_ih_1, b_hh_1, W_self_1, b_self_1, W_neigh_1):
    idx0 = nbr_0.astype(jnp.int32).T.reshape(-1)
    idx1 = nbr_1.astype(jnp.int32).T.reshape(-1)
    f0_flat, f1_flat = _sc_gather(x, idx0, idx1)
    f0 = f0_flat.reshape(DEG, N, D)
    f1 = f1_flat.reshape(DEG, N, D)

    wcat0 = jnp.concatenate([W_ih_0, W_hh_0], axis=1).T
    wcat1 = jnp.concatenate([W_ih_1, W_hh_1], axis=1).T
    bc0 = (b_ih_0 + b_hh_0).reshape(1, 4 * D)
    bc1 = (b_ih_1 + b_hh_1).reshape(1, 4 * D)

    out = pl.pallas_call(
        _tc_body,
        grid=(_GRID,),
        in_specs=[
            pl.BlockSpec((_BLK, D), lambda i: (i, 0)),
            pl.BlockSpec((DEG, _BLK, D), lambda i: (0, i, 0)),
            pl.BlockSpec((DEG, _BLK, D), lambda i: (0, i, 0)),
            _full((2 * D, 4 * D)), _full((1, 4 * D)),
            _full((D, D)), _full((1, D)), _full((D, D)),
            _full((2 * D, 4 * D)), _full((1, 4 * D)),
            _full((D, D)), _full((1, D)), _full((D, D)),
            _full((D, D)),
        ],
        out_specs=pl.BlockSpec((_BLK, D), lambda i: (i, 0)),
        out_shape=jax.ShapeDtypeStruct((N, D), jnp.float32),
    )(x, f0, f1,
      wcat0, bc0, W_self_0.T, b_self_0.reshape(1, D), W_neigh_0.T,
      wcat1, bc1, W_self_1.T, b_self_1.reshape(1, D), W_neigh_1.T,
      trans)
    return out
